# Initial kernel scaffold; baseline (speedup 1.0000x reference)
import jax
import jax.numpy as jnp
from jax import lax
from jax.experimental import pallas as pl
from jax.experimental.pallas import tpu as pltpu
from jax.experimental.pallas import tpu_sc as plsc

B = 4
N = 12500
NP = 12800
E = 200000
EP = 204800
H = 64
H2 = 32
NS = 30
NC = 300
NR = 16

BN = 512
NB = NP // BN

NCORES = 2
NSUB = 16
NW = NCORES * NSUB
EPW = EP // NW
C = 128
NCHUNK = EPW // C
ROWS_PER_SUB = NP // NSUB
DUMMY_DST = 12600

_F32 = jnp.float32


def _mlp_einsum_body(cls_ref, st_ref, cemb_ref, swt_ref, sb_ref,
                     w1t_ref, b1_ref, w2t_ref, b2_ref, wcat_ref, out_ref):
    cls = cls_ref[0]
    iota = lax.broadcasted_iota(_F32, (BN, NC), 1)
    onehot = (iota == cls).astype(_F32)
    cn = jnp.dot(onehot, cemb_ref[...], preferred_element_type=_F32)
    se = jnp.dot(st_ref[0], swt_ref[...], preferred_element_type=_F32)
    se = se + sb_ref[...]
    h = jax.nn.relu(jnp.concatenate([cn, se], axis=1))
    h = jax.nn.relu(jnp.dot(h, w1t_ref[...], preferred_element_type=_F32)
                    + b1_ref[...])
    h = jax.nn.relu(jnp.dot(h, w2t_ref[...], preferred_element_type=_F32)
                    + b2_ref[...])
    hw = jnp.dot(h, wcat_ref[...], preferred_element_type=_F32)
    for r in range(NR):
        out_ref[0, r] = hw[:, r * H:(r + 1) * H]


def _mlp_einsum(cls_pad, states_pad, class_emb, swt, sb, w1t, b1, w2t, b2, wcat):
    full = lambda shape: pl.BlockSpec(shape, lambda b, i: (0,) * len(shape))
    return pl.pallas_call(
        _mlp_einsum_body,
        grid=(B, NB),
        in_specs=[
            pl.BlockSpec((1, BN, 1), lambda b, i: (b, i, 0)),
            pl.BlockSpec((1, BN, NS), lambda b, i: (b, i, 0)),
            full((NC, H2)), full((NS, H2)), full((1, H2)),
            full((H, H)), full((1, H)), full((H, H)), full((1, H)),
            full((H, NR * H)),
        ],
        out_specs=pl.BlockSpec((1, NR, BN, H), lambda b, i: (b, 0, i, 0)),
        out_shape=jax.ShapeDtypeStruct((B, NR, NP, H), _F32),
    )(cls_pad, states_pad, class_emb, swt, sb, w1t, b1, w2t, b2, wcat)


def _combine_einsum_body(p_ref, wcat_ref, out_ref):
    h = jax.nn.relu(p_ref[0, 0] + p_ref[0, 1])
    hw = jnp.dot(h, wcat_ref[...], preferred_element_type=_F32)
    for r in range(NR):
        out_ref[0, r] = hw[:, r * H:(r + 1) * H]


def _combine_einsum(p, wcat):
    return pl.pallas_call(
        _combine_einsum_body,
        grid=(B, NB),
        in_specs=[
            pl.BlockSpec((1, 2, BN, H), lambda b, i: (b, 0, i, 0)),
            pl.BlockSpec((H, NR * H), lambda b, i: (0, 0)),
        ],
        out_specs=pl.BlockSpec((1, NR, BN, H), lambda b, i: (b, 0, i, 0)),
        out_shape=jax.ShapeDtypeStruct((B, NR, NP, H), _F32),
    )(p, wcat)


def _combine_body(p_ref, out_ref):
    out_ref[0] = jax.nn.relu(p_ref[0, 0] + p_ref[0, 1])


def _combine(p):
    return pl.pallas_call(
        _combine_body,
        grid=(B, NB),
        in_specs=[pl.BlockSpec((1, 2, BN, H), lambda b, i: (b, 0, i, 0))],
        out_specs=pl.BlockSpec((1, BN, H), lambda b, i: (b, i, 0)),
        out_shape=jax.ShapeDtypeStruct((B, NP, H), _F32),
    )(p)


def _edge_agg_body(gidx_hbm, dst_hbm, hw_hbm, zeros_hbm, out_hbm,
                   idx_v, dst_v, rows_v, agg, sem):
    cid = lax.axis_index("c")
    sid = lax.axis_index("s")
    wid = sid * NCORES + cid
    my_rows = pl.multiple_of(sid * ROWS_PER_SUB, 8)
    for b in range(B):
        pltpu.sync_copy(zeros_hbm, agg.at[pl.ds(my_rows, ROWS_PER_SUB)])
        plsc.subcore_barrier()

        def chunk(j, carry):
            base = pl.multiple_of(b * EP + wid * EPW + j * C, C)
            pltpu.sync_copy(gidx_hbm.at[pl.ds(base, C)], idx_v)
            pltpu.sync_copy(dst_hbm.at[pl.ds(base, C)], dst_v)
            pltpu.async_copy(hw_hbm.at[idx_v], rows_v, sem).wait()
            pltpu.sync_copy(rows_v, agg.at[dst_v], add=True)
            return carry

        lax.fori_loop(0, NCHUNK, chunk, 0)
        plsc.subcore_barrier()
        off = pl.multiple_of(((b * NCORES + cid) * NP) + sid * ROWS_PER_SUB, 8)
        pltpu.sync_copy(agg.at[pl.ds(my_rows, ROWS_PER_SUB)],
                        out_hbm.at[pl.ds(off, ROWS_PER_SUB)])


def _edge_agg(gidx, dst, hw_flat, zeros):
    mesh = plsc.VectorSubcoreMesh(core_axis_name="c", subcore_axis_name="s",
                                  num_cores=NCORES, num_subcores=NSUB)
    k = pl.kernel(
        _edge_agg_body,
        out_type=jax.ShapeDtypeStruct((B * NCORES * NP, H), _F32),
        mesh=mesh,
        scratch_types=[
            pltpu.VMEM((C,), jnp.int32),
            pltpu.VMEM((C,), jnp.int32),
            pltpu.VMEM((C, H), _F32),
            pltpu.VMEM_SHARED((NP, H), _F32),
            pltpu.SemaphoreType.DMA,
        ],
    )
    return k(gidx, dst, hw_flat, zeros)


def kernel(class_objects, states_objects, edge_tuples, edge_classes,
           mask_object, mask_edge, class_emb, state_W, state_b,
           W1, b1, W2, b2, rgcn_W0, rgcn_W1):
    src = edge_tuples[:, :, 0].astype(jnp.int32)
    dst = edge_tuples[:, :, 1].astype(jnp.int32)
    rel = edge_classes.astype(jnp.int32)
    boff = (jnp.arange(B, dtype=jnp.int32) * (NR * NP))[:, None]
    gidx = rel * NP + src + boff
    gidx_pad = jnp.concatenate(
        [gidx, jnp.broadcast_to(boff, (B, EP - E))], axis=1).reshape(-1)
    dst_pad = jnp.concatenate(
        [dst, jnp.full((B, EP - E), DUMMY_DST, jnp.int32)], axis=1).reshape(-1)

    cls_pad = jnp.pad(class_objects.astype(_F32), ((0, 0), (0, NP - N)))
    cls_pad = cls_pad[:, :, None]
    states_pad = jnp.pad(states_objects, ((0, 0), (0, NP - N), (0, 0)))

    swt = state_W.T
    sb = state_b[None, :]
    w1t, w2t = W1.T, W2.T
    b1r, b2r = b1[None, :], b2[None, :]
    wcat0 = rgcn_W0.transpose(1, 0, 2).reshape(H, NR * H)
    wcat1 = rgcn_W1.transpose(1, 0, 2).reshape(H, NR * H)
    zeros = jnp.zeros((ROWS_PER_SUB, H), _F32)

    hw1 = _mlp_einsum(cls_pad, states_pad, class_emb, swt, sb,
                      w1t, b1r, w2t, b2r, wcat0)
    p1 = _edge_agg(gidx_pad, dst_pad, hw1.reshape(B * NR * NP, H), zeros)
    hw2 = _combine_einsum(p1.reshape(B, NCORES, NP, H), wcat1)
    p2 = _edge_agg(gidx_pad, dst_pad, hw2.reshape(B * NR * NP, H), zeros)
    out = _combine(p2.reshape(B, NCORES, NP, H))
    return out[:, :N, :]


# speedup vs baseline: 1.6021x; 1.6021x over previous
import jax
import jax.numpy as jnp
from jax import lax
from jax.experimental import pallas as pl
from jax.experimental.pallas import tpu as pltpu
from jax.experimental.pallas import tpu_sc as plsc

B = 4
N = 12500
NP = 12800
E = 200000
EP = 204800
H = 64
H2 = 32
NS = 30
NC = 300
NR = 16

BN = 512
NB = NP // BN

NCORES = 2
NSUB = 16
NW = NCORES * NSUB
EPW = EP // NW
C = 128
NCHUNK = EPW // C
ROWS_PER_SUB = NP // NSUB
DUMMY_DST = 12600

_F32 = jnp.float32


def _mlp_einsum_body(cls_ref, st_ref, cemb_ref, swt_ref, sb_ref,
                     w1t_ref, b1_ref, w2t_ref, b2_ref, wcat_ref, out_ref):
    cls = cls_ref[0].astype(jnp.int32)
    iota = lax.broadcasted_iota(jnp.int32, (BN, NC), 1)
    onehot = (iota == cls).astype(_F32)
    cn = jnp.dot(onehot, cemb_ref[...], preferred_element_type=_F32)
    se = jnp.dot(st_ref[0], swt_ref[...], preferred_element_type=_F32)
    se = se + sb_ref[...]
    h = jax.nn.relu(jnp.concatenate([cn, se], axis=1))
    h = jax.nn.relu(jnp.dot(h, w1t_ref[...], preferred_element_type=_F32)
                    + b1_ref[...])
    h = jax.nn.relu(jnp.dot(h, w2t_ref[...], preferred_element_type=_F32)
                    + b2_ref[...])
    hw = jnp.dot(h, wcat_ref[...], preferred_element_type=_F32)
    for r in range(NR):
        out_ref[0, r] = hw[:, r * H:(r + 1) * H]


def _mlp_einsum(cls_pad, states_pad, class_emb, swt, sb, w1t, b1, w2t, b2, wcat):
    full = lambda shape: pl.BlockSpec(shape, lambda b, i: (0,) * len(shape))
    return pl.pallas_call(
        _mlp_einsum_body,
        grid=(B, NB),
        in_specs=[
            pl.BlockSpec((1, BN, 1), lambda b, i: (b, i, 0)),
            pl.BlockSpec((1, BN, NS), lambda b, i: (b, i, 0)),
            full((NC, H2)), full((NS, H2)), full((1, H2)),
            full((H, H)), full((1, H)), full((H, H)), full((1, H)),
            full((H, NR * H)),
        ],
        out_specs=pl.BlockSpec((1, NR, BN, H), lambda b, i: (b, 0, i, 0)),
        out_shape=jax.ShapeDtypeStruct((B, NR, NP, H), _F32),
    )(cls_pad, states_pad, class_emb, swt, sb, w1t, b1, w2t, b2, wcat)


def _combine_einsum_body(p_ref, wcat_ref, out_ref):
    h = jax.nn.relu(p_ref[0, 0] + p_ref[0, 1])
    hw = jnp.dot(h, wcat_ref[...], preferred_element_type=_F32)
    for r in range(NR):
        out_ref[0, r] = hw[:, r * H:(r + 1) * H]


def _combine_einsum(p, wcat):
    return pl.pallas_call(
        _combine_einsum_body,
        grid=(B, NB),
        in_specs=[
            pl.BlockSpec((1, 2, BN, H), lambda b, i: (b, 0, i, 0)),
            pl.BlockSpec((H, NR * H), lambda b, i: (0, 0)),
        ],
        out_specs=pl.BlockSpec((1, NR, BN, H), lambda b, i: (b, 0, i, 0)),
        out_shape=jax.ShapeDtypeStruct((B, NR, NP, H), _F32),
    )(p, wcat)


def _combine_body(p_ref, out_ref):
    out_ref[0] = jax.nn.relu(p_ref[0, 0] + p_ref[0, 1])


def _combine(p):
    return pl.pallas_call(
        _combine_body,
        grid=(B, NB),
        in_specs=[pl.BlockSpec((1, 2, BN, H), lambda b, i: (b, 0, i, 0))],
        out_specs=pl.BlockSpec((1, BN, H), lambda b, i: (b, i, 0)),
        out_shape=jax.ShapeDtypeStruct((B, NP, H), _F32),
    )(p)


def _edge_agg_body(gidx_hbm, dst_hbm, hw_hbm, zeros_hbm, out_hbm,
                   idx_v, dst_v, rows_v, agg, sem):
    cid = lax.axis_index("c")
    sid = lax.axis_index("s")
    wid = sid * NCORES + cid
    my_rows = pl.multiple_of(sid * ROWS_PER_SUB, 8)
    for b in range(B):
        pltpu.sync_copy(zeros_hbm, agg.at[pl.ds(my_rows, ROWS_PER_SUB)])
        plsc.subcore_barrier()

        def chunk(j, carry):
            base = pl.multiple_of(b * EP + wid * EPW + j * C, C)
            pltpu.sync_copy(gidx_hbm.at[pl.ds(base, C)], idx_v)
            pltpu.sync_copy(dst_hbm.at[pl.ds(base, C)], dst_v)
            pltpu.async_copy(hw_hbm.at[idx_v], rows_v, sem).wait()
            pltpu.sync_copy(rows_v, agg.at[dst_v], add=True)
            return carry

        lax.fori_loop(0, NCHUNK, chunk, 0)
        plsc.subcore_barrier()
        off = pl.multiple_of(((b * NCORES + cid) * NP) + sid * ROWS_PER_SUB, 8)
        pltpu.sync_copy(agg.at[pl.ds(my_rows, ROWS_PER_SUB)],
                        out_hbm.at[pl.ds(off, ROWS_PER_SUB)])


def _edge_agg(gidx, dst, hw_flat, zeros):
    mesh = plsc.VectorSubcoreMesh(core_axis_name="c", subcore_axis_name="s",
                                  num_cores=NCORES, num_subcores=NSUB)
    k = pl.kernel(
        _edge_agg_body,
        out_type=jax.ShapeDtypeStruct((B * NCORES * NP, H), _F32),
        mesh=mesh,
        scratch_types=[
            pltpu.VMEM((C,), jnp.int32),
            pltpu.VMEM((C,), jnp.int32),
            pltpu.VMEM((C, H), _F32),
            pltpu.VMEM_SHARED((NP, H), _F32),
            pltpu.SemaphoreType.DMA,
        ],
        compiler_params=pltpu.CompilerParams(use_tc_tiling_on_sc=False),
    )
    return k(gidx, dst, hw_flat, zeros)


def kernel(class_objects, states_objects, edge_tuples, edge_classes,
           mask_object, mask_edge, class_emb, state_W, state_b,
           W1, b1, W2, b2, rgcn_W0, rgcn_W1):
    src = edge_tuples[:, :, 0].astype(jnp.int32)
    dst = edge_tuples[:, :, 1].astype(jnp.int32)
    rel = edge_classes.astype(jnp.int32)
    boff = (jnp.arange(B, dtype=jnp.int32) * (NR * NP))[:, None]
    gidx = rel * NP + src + boff
    gidx_pad = jnp.concatenate(
        [gidx, jnp.broadcast_to(boff, (B, EP - E))], axis=1).reshape(-1)
    dst_pad = jnp.concatenate(
        [dst, jnp.full((B, EP - E), DUMMY_DST, jnp.int32)], axis=1).reshape(-1)

    cls_pad = jnp.pad(class_objects.astype(_F32), ((0, 0), (0, NP - N)))
    cls_pad = cls_pad[:, :, None]
    states_pad = jnp.pad(states_objects, ((0, 0), (0, NP - N), (0, 0)))

    swt = state_W.T
    sb = state_b[None, :]
    w1t, w2t = W1.T, W2.T
    b1r, b2r = b1[None, :], b2[None, :]
    wcat0 = rgcn_W0.transpose(1, 0, 2).reshape(H, NR * H)
    wcat1 = rgcn_W1.transpose(1, 0, 2).reshape(H, NR * H)
    zeros = jnp.zeros((ROWS_PER_SUB, H), _F32)

    hw1 = _mlp_einsum(cls_pad, states_pad, class_emb, swt, sb,
                      w1t, b1r, w2t, b2r, wcat0)
    p1 = _edge_agg(gidx_pad, dst_pad, hw1.reshape(B * NR * NP, H), zeros)
    hw2 = _combine_einsum(p1.reshape(B, NCORES, NP, H), wcat1)
    p2 = _edge_agg(gidx_pad, dst_pad, hw2.reshape(B * NR * NP, H), zeros)
    out = _combine(p2.reshape(B, NCORES, NP, H))
    return out[:, :N, :]


# speedup vs baseline: 1.9813x; 1.2367x over previous
import jax
import jax.numpy as jnp
from jax import lax
from jax.experimental import pallas as pl
from jax.experimental.pallas import tpu as pltpu
from jax.experimental.pallas import tpu_sc as plsc

B = 4
N = 12500
NP = 12800
E = 200000
EP = 204800
H = 64
H2 = 32
NS = 30
NC = 300
NR = 16

BN = 512
NB = NP // BN

NCORES = 2
NSUB = 16
NW = NCORES * NSUB
EPW = EP // NW
C = 80
NCHUNK = EPW // C
ROWS_PER_SUB = NP // NSUB
DUMMY_DST = 12600

_F32 = jnp.float32


def _mlp_einsum_body(cls_ref, st_ref, cemb_ref, swt_ref, sb_ref,
                     w1t_ref, b1_ref, w2t_ref, b2_ref, wcat_ref, out_ref):
    cls = cls_ref[0].astype(jnp.int32)
    iota = lax.broadcasted_iota(jnp.int32, (BN, NC), 1)
    onehot = (iota == cls).astype(_F32)
    cn = jnp.dot(onehot, cemb_ref[...], preferred_element_type=_F32)
    se = jnp.dot(st_ref[0], swt_ref[...], preferred_element_type=_F32)
    se = se + sb_ref[...]
    h = jax.nn.relu(jnp.concatenate([cn, se], axis=1))
    h = jax.nn.relu(jnp.dot(h, w1t_ref[...], preferred_element_type=_F32)
                    + b1_ref[...])
    h = jax.nn.relu(jnp.dot(h, w2t_ref[...], preferred_element_type=_F32)
                    + b2_ref[...])
    hw = jnp.dot(h, wcat_ref[...], preferred_element_type=_F32)
    for r in range(NR):
        out_ref[0, r] = hw[:, r * H:(r + 1) * H]


def _mlp_einsum(cls_pad, states_pad, class_emb, swt, sb, w1t, b1, w2t, b2, wcat):
    full = lambda shape: pl.BlockSpec(shape, lambda b, i: (0,) * len(shape))
    return pl.pallas_call(
        _mlp_einsum_body,
        grid=(B, NB),
        in_specs=[
            pl.BlockSpec((1, BN, 1), lambda b, i: (b, i, 0)),
            pl.BlockSpec((1, BN, NS), lambda b, i: (b, i, 0)),
            full((NC, H2)), full((NS, H2)), full((1, H2)),
            full((H, H)), full((1, H)), full((H, H)), full((1, H)),
            full((H, NR * H)),
        ],
        out_specs=pl.BlockSpec((1, NR, BN, H), lambda b, i: (b, 0, i, 0)),
        out_shape=jax.ShapeDtypeStruct((B, NR, NP, H), _F32),
    )(cls_pad, states_pad, class_emb, swt, sb, w1t, b1, w2t, b2, wcat)


def _combine_einsum_body(p_ref, wcat_ref, out_ref):
    h = jax.nn.relu(p_ref[0, 0] + p_ref[0, 1])
    hw = jnp.dot(h, wcat_ref[...], preferred_element_type=_F32)
    for r in range(NR):
        out_ref[0, r] = hw[:, r * H:(r + 1) * H]


def _combine_einsum(p, wcat):
    return pl.pallas_call(
        _combine_einsum_body,
        grid=(B, NB),
        in_specs=[
            pl.BlockSpec((1, 2, BN, H), lambda b, i: (b, 0, i, 0)),
            pl.BlockSpec((H, NR * H), lambda b, i: (0, 0)),
        ],
        out_specs=pl.BlockSpec((1, NR, BN, H), lambda b, i: (b, 0, i, 0)),
        out_shape=jax.ShapeDtypeStruct((B, NR, NP, H), _F32),
    )(p, wcat)


def _combine_body(p_ref, out_ref):
    out_ref[0] = jax.nn.relu(p_ref[0, 0] + p_ref[0, 1])


def _combine(p):
    return pl.pallas_call(
        _combine_body,
        grid=(B, NB),
        in_specs=[pl.BlockSpec((1, 2, BN, H), lambda b, i: (b, 0, i, 0))],
        out_specs=pl.BlockSpec((1, BN, H), lambda b, i: (b, i, 0)),
        out_shape=jax.ShapeDtypeStruct((B, NP, H), _F32),
    )(p)


RBUF = 10
DEPTH = 5
NOUT = NCHUNK // RBUF


def _edge_agg_body(gidx_hbm, dst_hbm, hw_hbm, zeros_hbm, out_hbm,
                   idx_v, dst_v, rows, agg, *sems):
    gsem, ssem = sems[:RBUF], sems[RBUF:]
    cid = lax.axis_index("c")
    sid = lax.axis_index("s")
    wid = sid * NCORES + cid
    my_rows = pl.multiple_of(sid * ROWS_PER_SUB, 8)

    def wait_gather(c, k):
        pltpu.make_async_copy(hw_hbm.at[idx_v.at[c]], rows.at[k],
                              gsem[k]).wait()

    def wait_scatter(c, k):
        pltpu.make_async_copy(rows.at[k], agg.at[dst_v.at[c]],
                              ssem[k]).wait()

    for b in range(B):
        pltpu.sync_copy(zeros_hbm, agg.at[pl.ds(my_rows, ROWS_PER_SUB)])
        pltpu.sync_copy(gidx_hbm.at[b * NW + wid], idx_v)
        pltpu.sync_copy(dst_hbm.at[b * NW + wid], dst_v)
        plsc.subcore_barrier()

        for c in range(DEPTH):
            pltpu.async_copy(hw_hbm.at[idx_v.at[c]], rows.at[c % RBUF],
                             gsem[c % RBUF])

        def outer(jj, carry):
            for k in range(RBUF):
                c = jj * RBUF + k
                wait_gather(c, k)
                pltpu.async_copy(rows.at[k], agg.at[dst_v.at[c]],
                                 ssem[k], add=True)
                n = c + DEPTH
                kn = (k + DEPTH) % RBUF
                if k < RBUF - DEPTH:
                    @pl.when(jj >= 1)
                    def _():
                        wait_scatter(n - RBUF, kn)
                    pltpu.async_copy(hw_hbm.at[idx_v.at[n]], rows.at[kn],
                                     gsem[kn])
                else:
                    @pl.when(jj < NOUT - 1)
                    def _():
                        wait_scatter(n - RBUF, kn)
                        pltpu.async_copy(hw_hbm.at[idx_v.at[n]], rows.at[kn],
                                         gsem[kn])
            return carry

        lax.fori_loop(0, NOUT, outer, 0)
        for c in range(NCHUNK - DEPTH, NCHUNK):
            wait_scatter(c, c % RBUF)
        plsc.subcore_barrier()
        off = pl.multiple_of(((b * NCORES + cid) * NP) + sid * ROWS_PER_SUB, 8)
        pltpu.sync_copy(agg.at[pl.ds(my_rows, ROWS_PER_SUB)],
                        out_hbm.at[pl.ds(off, ROWS_PER_SUB)])


def _edge_agg(gidx, dst, hw_flat, zeros):
    mesh = plsc.VectorSubcoreMesh(core_axis_name="c", subcore_axis_name="s",
                                  num_cores=NCORES, num_subcores=NSUB)
    k = pl.kernel(
        _edge_agg_body,
        out_type=jax.ShapeDtypeStruct((B * NCORES * NP, H), _F32),
        mesh=mesh,
        scratch_types=[
            pltpu.VMEM((NCHUNK, C), jnp.int32),
            pltpu.VMEM((NCHUNK, C), jnp.int32),
            pltpu.VMEM((RBUF, C, H), _F32),
            pltpu.VMEM_SHARED((NP, H), _F32),
        ] + [pltpu.SemaphoreType.DMA] * (2 * RBUF),
        compiler_params=pltpu.CompilerParams(use_tc_tiling_on_sc=False),
    )
    return k(gidx, dst, hw_flat, zeros)


def kernel(class_objects, states_objects, edge_tuples, edge_classes,
           mask_object, mask_edge, class_emb, state_W, state_b,
           W1, b1, W2, b2, rgcn_W0, rgcn_W1):
    src = edge_tuples[:, :, 0].astype(jnp.int32)
    dst = edge_tuples[:, :, 1].astype(jnp.int32)
    rel = edge_classes.astype(jnp.int32)
    boff = (jnp.arange(B, dtype=jnp.int32) * (NR * NP))[:, None]
    gidx = rel * NP + src + boff
    gidx_pad = jnp.concatenate(
        [gidx, jnp.broadcast_to(boff, (B, EP - E))], axis=1).reshape(-1)
    dst_pad = jnp.concatenate(
        [dst, jnp.full((B, EP - E), DUMMY_DST, jnp.int32)], axis=1).reshape(-1)

    cls_pad = jnp.pad(class_objects.astype(_F32), ((0, 0), (0, NP - N)))
    cls_pad = cls_pad[:, :, None]
    states_pad = jnp.pad(states_objects, ((0, 0), (0, NP - N), (0, 0)))

    swt = state_W.T
    sb = state_b[None, :]
    w1t, w2t = W1.T, W2.T
    b1r, b2r = b1[None, :], b2[None, :]
    wcat0 = rgcn_W0.transpose(1, 0, 2).reshape(H, NR * H)
    wcat1 = rgcn_W1.transpose(1, 0, 2).reshape(H, NR * H)
    zeros = jnp.zeros((ROWS_PER_SUB, H), _F32)

    gidx_pad = gidx_pad.reshape(B * NW, NCHUNK, C)
    dst_pad = dst_pad.reshape(B * NW, NCHUNK, C)

    hw1 = _mlp_einsum(cls_pad, states_pad, class_emb, swt, sb,
                      w1t, b1r, w2t, b2r, wcat0)
    p1 = _edge_agg(gidx_pad, dst_pad, hw1.reshape(B * NR * NP, H), zeros)
    hw2 = _combine_einsum(p1.reshape(B, NCORES, NP, H), wcat1)
    p2 = _edge_agg(gidx_pad, dst_pad, hw2.reshape(B * NR * NP, H), zeros)
    out = _combine(p2.reshape(B, NCORES, NP, H))
    return out[:, :N, :]


# speedup vs baseline: 2.0528x; 1.0361x over previous
import jax
import jax.numpy as jnp
from jax import lax
from jax.experimental import pallas as pl
from jax.experimental.pallas import tpu as pltpu
from jax.experimental.pallas import tpu_sc as plsc

B = 4
N = 12500
NP = 12800
E = 200000
EP = 204800
H = 64
H2 = 32
NS = 30
NC = 300
NR = 16

BN = 1280
NB = NP // BN

NCORES = 2
NSUB = 16
NW = NCORES * NSUB
EPW = EP // NW
C = 80
NCHUNK = EPW // C
ROWS_PER_SUB = NP // NSUB
DUMMY_DST = 12600

_F32 = jnp.float32


def _mlp_einsum_body(cls_ref, st_ref, cemb_ref, swt_ref, sb_ref,
                     w1t_ref, b1_ref, w2t_ref, b2_ref, wcat_ref, out_ref):
    cls = cls_ref[0].astype(jnp.int32)
    iota = lax.broadcasted_iota(jnp.int32, (BN, NC), 1)
    onehot = (iota == cls).astype(_F32)
    cn = jnp.dot(onehot, cemb_ref[...], preferred_element_type=_F32)
    se = jnp.dot(st_ref[0], swt_ref[...], preferred_element_type=_F32)
    se = se + sb_ref[...]
    h = jax.nn.relu(jnp.concatenate([cn, se], axis=1))
    h = jax.nn.relu(jnp.dot(h, w1t_ref[...], preferred_element_type=_F32)
                    + b1_ref[...])
    h = jax.nn.relu(jnp.dot(h, w2t_ref[...], preferred_element_type=_F32)
                    + b2_ref[...])
    hw = jnp.dot(h, wcat_ref[...], preferred_element_type=_F32)
    for r in range(NR):
        out_ref[r * BN:(r + 1) * BN, :] = hw[:, r * H:(r + 1) * H]


def _mlp_einsum(cls_pad, states_pad, class_emb, swt, sb, w1t, b1, w2t, b2, wcat):
    full = lambda shape: pl.BlockSpec(shape, lambda b, i: (0,) * len(shape))
    return pl.pallas_call(
        _mlp_einsum_body,
        grid=(B, NB),
        in_specs=[
            pl.BlockSpec((1, BN, 1), lambda b, i: (b, i, 0)),
            pl.BlockSpec((1, BN, NS), lambda b, i: (b, i, 0)),
            full((NC, H2)), full((NS, H2)), full((1, H2)),
            full((H, H)), full((1, H)), full((H, H)), full((1, H)),
            full((H, NR * H)),
        ],
        out_specs=pl.BlockSpec((NR * BN, H), lambda b, i: (b * NB + i, 0)),
        out_shape=jax.ShapeDtypeStruct((B * NP * NR, H), _F32),
    )(cls_pad, states_pad, class_emb, swt, sb, w1t, b1, w2t, b2, wcat)


def _combine_einsum_body(p_ref, wcat_ref, out_ref):
    h = jax.nn.relu(p_ref[0, 0] + p_ref[0, 1])
    hw = jnp.dot(h, wcat_ref[...], preferred_element_type=_F32)
    for r in range(NR):
        out_ref[r * BN:(r + 1) * BN, :] = hw[:, r * H:(r + 1) * H]


def _combine_einsum(p, wcat):
    return pl.pallas_call(
        _combine_einsum_body,
        grid=(B, NB),
        in_specs=[
            pl.BlockSpec((1, 2, BN, H), lambda b, i: (b, 0, i, 0)),
            pl.BlockSpec((H, NR * H), lambda b, i: (0, 0)),
        ],
        out_specs=pl.BlockSpec((NR * BN, H), lambda b, i: (b * NB + i, 0)),
        out_shape=jax.ShapeDtypeStruct((B * NP * NR, H), _F32),
    )(p, wcat)


def _combine_body(p_ref, out_ref):
    out_ref[0] = jax.nn.relu(p_ref[0, 0] + p_ref[0, 1])


def _combine(p):
    return pl.pallas_call(
        _combine_body,
        grid=(B, NB),
        in_specs=[pl.BlockSpec((1, 2, BN, H), lambda b, i: (b, 0, i, 0))],
        out_specs=pl.BlockSpec((1, BN, H), lambda b, i: (b, i, 0)),
        out_shape=jax.ShapeDtypeStruct((B, NP, H), _F32),
    )(p)


RBUF = 10
DEPTH = 5
NOUT = NCHUNK // RBUF


def _edge_agg_body(gidx_hbm, dst_hbm, hw_hbm, zeros_hbm, out_hbm,
                   idx_v, dst_v, rows, agg, *sems):
    gsem, ssem = sems[:RBUF], sems[RBUF:]
    cid = lax.axis_index("c")
    sid = lax.axis_index("s")
    wid = sid * NCORES + cid
    my_rows = pl.multiple_of(sid * ROWS_PER_SUB, 8)

    def wait_gather(c, k):
        pltpu.make_async_copy(hw_hbm.at[idx_v.at[c]], rows.at[k],
                              gsem[k]).wait()

    def wait_scatter(c, k):
        pltpu.make_async_copy(rows.at[k], agg.at[dst_v.at[c]],
                              ssem[k]).wait()

    for b in range(B):
        pltpu.sync_copy(zeros_hbm, agg.at[pl.ds(my_rows, ROWS_PER_SUB)])
        pltpu.sync_copy(gidx_hbm.at[b * NW + wid], idx_v)
        pltpu.sync_copy(dst_hbm.at[b * NW + wid], dst_v)
        plsc.subcore_barrier()

        for c in range(DEPTH):
            pltpu.async_copy(hw_hbm.at[idx_v.at[c]], rows.at[c % RBUF],
                             gsem[c % RBUF])

        def outer(jj, carry):
            for k in range(RBUF):
                c = jj * RBUF + k
                wait_gather(c, k)
                pltpu.async_copy(rows.at[k], agg.at[dst_v.at[c]],
                                 ssem[k], add=True)
                n = c + DEPTH
                kn = (k + DEPTH) % RBUF
                if k < RBUF - DEPTH:
                    @pl.when(jj >= 1)
                    def _():
                        wait_scatter(n - RBUF, kn)
                    pltpu.async_copy(hw_hbm.at[idx_v.at[n]], rows.at[kn],
                                     gsem[kn])
                else:
                    @pl.when(jj < NOUT - 1)
                    def _():
                        wait_scatter(n - RBUF, kn)
                        pltpu.async_copy(hw_hbm.at[idx_v.at[n]], rows.at[kn],
                                         gsem[kn])
            return carry

        lax.fori_loop(0, NOUT, outer, 0)
        for c in range(NCHUNK - DEPTH, NCHUNK):
            wait_scatter(c, c % RBUF)
        plsc.subcore_barrier()
        off = pl.multiple_of(((b * NCORES + cid) * NP) + sid * ROWS_PER_SUB, 8)
        pltpu.sync_copy(agg.at[pl.ds(my_rows, ROWS_PER_SUB)],
                        out_hbm.at[pl.ds(off, ROWS_PER_SUB)])


def _edge_agg(gidx, dst, hw_flat, zeros):
    mesh = plsc.VectorSubcoreMesh(core_axis_name="c", subcore_axis_name="s",
                                  num_cores=NCORES, num_subcores=NSUB)
    k = pl.kernel(
        _edge_agg_body,
        out_type=jax.ShapeDtypeStruct((B * NCORES * NP, H), _F32),
        mesh=mesh,
        scratch_types=[
            pltpu.VMEM((NCHUNK, C), jnp.int32),
            pltpu.VMEM((NCHUNK, C), jnp.int32),
            pltpu.VMEM((RBUF, C, H), _F32),
            pltpu.VMEM_SHARED((NP, H), _F32),
        ] + [pltpu.SemaphoreType.DMA] * (2 * RBUF),
        compiler_params=pltpu.CompilerParams(use_tc_tiling_on_sc=False),
    )
    return k(gidx, dst, hw_flat, zeros)


def kernel(class_objects, states_objects, edge_tuples, edge_classes,
           mask_object, mask_edge, class_emb, state_W, state_b,
           W1, b1, W2, b2, rgcn_W0, rgcn_W1):
    src = edge_tuples[:, :, 0].astype(jnp.int32)
    dst = edge_tuples[:, :, 1].astype(jnp.int32)
    rel = edge_classes.astype(jnp.int32)
    boff = (jnp.arange(B, dtype=jnp.int32) * NB)[:, None]
    gidx = ((boff + src // BN) * NR + rel) * BN + src % BN
    pad_row = jnp.broadcast_to(boff * NR * BN, (B, EP - E))
    gidx_pad = jnp.concatenate([gidx, pad_row], axis=1).reshape(-1)
    dst_pad = jnp.concatenate(
        [dst, jnp.full((B, EP - E), DUMMY_DST, jnp.int32)], axis=1).reshape(-1)

    cls_pad = jnp.pad(class_objects.astype(_F32), ((0, 0), (0, NP - N)))
    cls_pad = cls_pad[:, :, None]
    states_pad = jnp.pad(states_objects, ((0, 0), (0, NP - N), (0, 0)))

    swt = state_W.T
    sb = state_b[None, :]
    w1t, w2t = W1.T, W2.T
    b1r, b2r = b1[None, :], b2[None, :]
    wcat0 = rgcn_W0.transpose(1, 0, 2).reshape(H, NR * H)
    wcat1 = rgcn_W1.transpose(1, 0, 2).reshape(H, NR * H)
    zeros = jnp.zeros((ROWS_PER_SUB, H), _F32)

    gidx_pad = gidx_pad.reshape(B * NW, NCHUNK, C)
    dst_pad = dst_pad.reshape(B * NW, NCHUNK, C)

    hw1 = _mlp_einsum(cls_pad, states_pad, class_emb, swt, sb,
                      w1t, b1r, w2t, b2r, wcat0)
    p1 = _edge_agg(gidx_pad, dst_pad, hw1, zeros)
    hw2 = _combine_einsum(p1.reshape(B, NCORES, NP, H), wcat1)
    p2 = _edge_agg(gidx_pad, dst_pad, hw2, zeros)
    out = _combine(p2.reshape(B, NCORES, NP, H))
    return out[:, :N, :]


# speedup vs baseline: 2.3214x; 1.1308x over previous
import jax
import jax.numpy as jnp
from jax import lax
from jax.experimental import pallas as pl
from jax.experimental.pallas import tpu as pltpu
from jax.experimental.pallas import tpu_sc as plsc

B = 4
N = 12500
NP = 12800
E = 200000
EPB = 204800
H = 64
H2 = 32
NS = 30
NC = 300
NR = 16

BN = 1280
NB = NP // BN

NCORES = 2
NSUB = 16
C = 80
CHUNKS_SID = 160
K0 = 125
K1 = CHUNKS_SID - K0
ROWS_PER_SUB = NP // NSUB
DUMMY_DST = 12600

RBUF = 10
DEPTH = 5

_F32 = jnp.float32


def _mlp_einsum_body(cls_ref, st_ref, cemb_ref, swt_ref, sb_ref,
                     w1t_ref, b1_ref, w2t_ref, b2_ref, wcat_ref, out_ref):
    cls = cls_ref[:, :].astype(jnp.int32)
    iota = lax.broadcasted_iota(jnp.int32, (BN, NC), 1)
    onehot = (iota == cls).astype(_F32)
    cn = jnp.dot(onehot, cemb_ref[...], preferred_element_type=_F32)
    se = jnp.dot(st_ref[...], swt_ref[...], preferred_element_type=_F32)
    se = se + sb_ref[...]
    h = jax.nn.relu(jnp.concatenate([cn, se], axis=1))
    h = jax.nn.relu(jnp.dot(h, w1t_ref[...], preferred_element_type=_F32)
                    + b1_ref[...])
    h = jax.nn.relu(jnp.dot(h, w2t_ref[...], preferred_element_type=_F32)
                    + b2_ref[...])
    hw = jnp.dot(h, wcat_ref[...], preferred_element_type=_F32)
    for r in range(NR):
        out_ref[r * BN:(r + 1) * BN, :] = hw[:, r * H:(r + 1) * H]


def _mlp_einsum(cls_b, states_b, class_emb, swt, sb, w1t, b1, w2t, b2, wcat):
    full = lambda shape: pl.BlockSpec(shape, lambda i: (0,) * len(shape))
    return pl.pallas_call(
        _mlp_einsum_body,
        grid=(NB,),
        in_specs=[
            pl.BlockSpec((BN, 1), lambda i: (i, 0)),
            pl.BlockSpec((BN, NS), lambda i: (i, 0)),
            full((NC, H2)), full((NS, H2)), full((1, H2)),
            full((H, H)), full((1, H)), full((H, H)), full((1, H)),
            full((H, NR * H)),
        ],
        out_specs=pl.BlockSpec((NR * BN, H), lambda i: (i, 0)),
        out_shape=jax.ShapeDtypeStruct((NP * NR, H), _F32),
    )(cls_b, states_b, class_emb, swt, sb, w1t, b1, w2t, b2, wcat)


def _combine_einsum_body(p_ref, wcat_ref, out_ref):
    h = jax.nn.relu(p_ref[0] + p_ref[1])
    hw = jnp.dot(h, wcat_ref[...], preferred_element_type=_F32)
    for r in range(NR):
        out_ref[r * BN:(r + 1) * BN, :] = hw[:, r * H:(r + 1) * H]


def _combine_einsum(p, wcat):
    return pl.pallas_call(
        _combine_einsum_body,
        grid=(NB,),
        in_specs=[
            pl.BlockSpec((2, BN, H), lambda i: (0, i, 0)),
            pl.BlockSpec((H, NR * H), lambda i: (0, 0)),
        ],
        out_specs=pl.BlockSpec((NR * BN, H), lambda i: (i, 0)),
        out_shape=jax.ShapeDtypeStruct((NP * NR, H), _F32),
    )(p, wcat)


def _combine_body(p_ref, out_ref):
    out_ref[...] = jax.nn.relu(p_ref[0] + p_ref[1])


def _combine(p):
    return pl.pallas_call(
        _combine_body,
        grid=(NB,),
        in_specs=[pl.BlockSpec((2, BN, H), lambda i: (0, i, 0))],
        out_specs=pl.BlockSpec((BN, H), lambda i: (i, 0)),
        out_shape=jax.ShapeDtypeStruct((NP, H), _F32),
    )(p)


def _edge_agg_body(gidx_hbm, dst_hbm, hw_hbm, zeros_hbm, out_hbm,
                   idx_v, dst_v, rows, agg, *sems):
    gsem, ssem = sems[:RBUF], sems[RBUF:]
    cid = lax.axis_index("c")
    sid = lax.axis_index("s")
    my_rows = pl.multiple_of(sid * ROWS_PER_SUB, 8)

    pltpu.sync_copy(zeros_hbm, agg.at[pl.ds(my_rows, ROWS_PER_SUB)])

    def run(base_chunk, k):
        pltpu.sync_copy(gidx_hbm.at[pl.ds(base_chunk, k)],
                        idx_v.at[pl.ds(0, k)])
        pltpu.sync_copy(dst_hbm.at[pl.ds(base_chunk, k)],
                        dst_v.at[pl.ds(0, k)])
        plsc.subcore_barrier()
        for c in range(DEPTH):
            pltpu.async_copy(hw_hbm.at[idx_v.at[c]], rows.at[c % RBUF],
                             gsem[c % RBUF])

        def group(jj, carry):
            for kk in range(RBUF):
                c = jj * RBUF + kk

                @pl.when(c < k)
                def _():
                    pltpu.make_async_copy(hw_hbm.at[idx_v.at[c]],
                                          rows.at[kk], gsem[kk]).wait()
                    pltpu.async_copy(rows.at[kk], agg.at[dst_v.at[c]],
                                     ssem[kk], add=True)
                    n = c + DEPTH
                    kn = (kk + DEPTH) % RBUF

                    @pl.when(n < k)
                    def _():
                        @pl.when(n >= RBUF)
                        def _():
                            pltpu.make_async_copy(
                                rows.at[kn], agg.at[dst_v.at[n - RBUF]],
                                ssem[kn]).wait()
                        pltpu.async_copy(hw_hbm.at[idx_v.at[n]],
                                         rows.at[kn], gsem[kn])
            return carry

        lax.fori_loop(0, (k + RBUF - 1) // RBUF, group, 0)
        for c in range(k - RBUF, k):
            pltpu.make_async_copy(rows.at[c % RBUF], agg.at[dst_v.at[c]],
                                  ssem[c % RBUF]).wait()

    @pl.when(cid == 0)
    def _():
        run(pl.multiple_of(sid * CHUNKS_SID, 8), K0)

    @pl.when(cid == 1)
    def _():
        run(sid * CHUNKS_SID + K0, K1)

    plsc.subcore_barrier()
    off = pl.multiple_of(cid * NP + sid * ROWS_PER_SUB, 8)
    pltpu.sync_copy(agg.at[pl.ds(my_rows, ROWS_PER_SUB)],
                    out_hbm.at[pl.ds(off, ROWS_PER_SUB)])


def _edge_agg(gidx, dst, hw_flat, zeros):
    mesh = plsc.VectorSubcoreMesh(core_axis_name="c", subcore_axis_name="s",
                                  num_cores=NCORES, num_subcores=NSUB)
    k = pl.kernel(
        _edge_agg_body,
        out_type=jax.ShapeDtypeStruct((NCORES * NP, H), _F32),
        mesh=mesh,
        scratch_types=[
            pltpu.VMEM((K0, C), jnp.int32),
            pltpu.VMEM((K0, C), jnp.int32),
            pltpu.VMEM((RBUF, C, H), _F32),
            pltpu.VMEM_SHARED((NP, H), _F32),
        ] + [pltpu.SemaphoreType.DMA] * (2 * RBUF),
        compiler_params=pltpu.CompilerParams(use_tc_tiling_on_sc=False),
    )
    return k(gidx, dst, hw_flat, zeros)


def kernel(class_objects, states_objects, edge_tuples, edge_classes,
           mask_object, mask_edge, class_emb, state_W, state_b,
           W1, b1, W2, b2, rgcn_W0, rgcn_W1):
    src = edge_tuples[:, :, 0].astype(jnp.int32)
    dst = edge_tuples[:, :, 1].astype(jnp.int32)
    rel = edge_classes.astype(jnp.int32)
    gidx = ((src // BN) * NR + rel) * BN + src % BN
    gidx_pad = jnp.concatenate(
        [gidx, jnp.zeros((B, EPB - E), jnp.int32)], axis=1)
    dst_pad = jnp.concatenate(
        [dst, jnp.full((B, EPB - E), DUMMY_DST, jnp.int32)], axis=1)
    gidx_pad = gidx_pad.reshape(B, NSUB * CHUNKS_SID, C)
    dst_pad = dst_pad.reshape(B, NSUB * CHUNKS_SID, C)

    cls_pad = jnp.pad(class_objects.astype(_F32), ((0, 0), (0, NP - N)))
    cls_pad = cls_pad[:, :, None]
    states_pad = jnp.pad(states_objects, ((0, 0), (0, NP - N), (0, 0)))

    swt = state_W.T
    sb = state_b[None, :]
    w1t, w2t = W1.T, W2.T
    b1r, b2r = b1[None, :], b2[None, :]
    wcat0 = rgcn_W0.transpose(1, 0, 2).reshape(H, NR * H)
    wcat1 = rgcn_W1.transpose(1, 0, 2).reshape(H, NR * H)
    zeros = jnp.zeros((ROWS_PER_SUB, H), _F32)

    outs = []
    for b in range(B):
        hw1 = _mlp_einsum(cls_pad[b], states_pad[b], class_emb, swt, sb,
                          w1t, b1r, w2t, b2r, wcat0)
        p1 = _edge_agg(gidx_pad[b], dst_pad[b], hw1, zeros)
        hw2 = _combine_einsum(p1.reshape(NCORES, NP, H), wcat1)
        p2 = _edge_agg(gidx_pad[b], dst_pad[b], hw2, zeros)
        outs.append(_combine(p2.reshape(NCORES, NP, H)))
    return jnp.stack(outs, axis=0)[:, :N, :]


# speedup vs baseline: 2.5657x; 1.1052x over previous
import jax
import jax.numpy as jnp
from jax import lax
from jax.experimental import pallas as pl
from jax.experimental.pallas import tpu as pltpu
from jax.experimental.pallas import tpu_sc as plsc

B = 4
N = 12500
NP = 12800
E = 200000
EPB = 204800
H = 64
H2 = 32
NS = 30
NC = 300
NR = 16

BN = 1280
NB = NP // BN

NCORES = 2
NSUB = 16
C = 80
CHUNKS_SID = 160
K0 = 125
K1 = CHUNKS_SID - K0
ROWS_PER_SUB = NP // NSUB
DUMMY_DST = 12600

RBUF = 10
DEPTH = 5

_F32 = jnp.float32
_BF16 = jnp.bfloat16


def _mlp_einsum_body(cls_ref, st_ref, cemb_ref, swt_ref, sb_ref,
                     w1t_ref, b1_ref, w2t_ref, b2_ref, wcat_ref, out_ref):
    cls = cls_ref[:, :].astype(jnp.int32)
    iota = lax.broadcasted_iota(jnp.int32, (BN, NC), 1)
    onehot = (iota == cls).astype(_F32)
    cn = jnp.dot(onehot, cemb_ref[...], preferred_element_type=_F32)
    se = jnp.dot(st_ref[...], swt_ref[...], preferred_element_type=_F32)
    se = se + sb_ref[...]
    h = jax.nn.relu(jnp.concatenate([cn, se], axis=1))
    h = jax.nn.relu(jnp.dot(h, w1t_ref[...], preferred_element_type=_F32)
                    + b1_ref[...])
    h = jax.nn.relu(jnp.dot(h, w2t_ref[...], preferred_element_type=_F32)
                    + b2_ref[...])
    hw = jnp.dot(h, wcat_ref[...],
                 preferred_element_type=_F32).astype(_BF16)
    for r in range(NR):
        out_ref[r * BN:(r + 1) * BN, :] = hw[:, r * H:(r + 1) * H]


def _mlp_einsum(cls_b, states_b, class_emb, swt, sb, w1t, b1, w2t, b2, wcat):
    full = lambda shape: pl.BlockSpec(shape, lambda i: (0,) * len(shape))
    return pl.pallas_call(
        _mlp_einsum_body,
        grid=(NB,),
        in_specs=[
            pl.BlockSpec((BN, 1), lambda i: (i, 0)),
            pl.BlockSpec((BN, NS), lambda i: (i, 0)),
            full((NC, H2)), full((NS, H2)), full((1, H2)),
            full((H, H)), full((1, H)), full((H, H)), full((1, H)),
            full((H, NR * H)),
        ],
        out_specs=pl.BlockSpec((NR * BN, H), lambda i: (i, 0)),
        out_shape=jax.ShapeDtypeStruct((NP * NR, H), _BF16),
    )(cls_b, states_b, class_emb, swt, sb, w1t, b1, w2t, b2, wcat)


def _combine_einsum_body(p_ref, wcat_ref, out_ref):
    h = jax.nn.relu(p_ref[0].astype(_F32) + p_ref[1].astype(_F32))
    hw = jnp.dot(h, wcat_ref[...],
                 preferred_element_type=_F32).astype(_BF16)
    for r in range(NR):
        out_ref[r * BN:(r + 1) * BN, :] = hw[:, r * H:(r + 1) * H]


def _combine_einsum(p, wcat):
    return pl.pallas_call(
        _combine_einsum_body,
        grid=(NB,),
        in_specs=[
            pl.BlockSpec((2, BN, H), lambda i: (0, i, 0)),
            pl.BlockSpec((H, NR * H), lambda i: (0, 0)),
        ],
        out_specs=pl.BlockSpec((NR * BN, H), lambda i: (i, 0)),
        out_shape=jax.ShapeDtypeStruct((NP * NR, H), _BF16),
    )(p, wcat)


def _combine_body(p_ref, out_ref):
    out_ref[...] = jax.nn.relu(p_ref[0].astype(_F32) + p_ref[1].astype(_F32))


def _combine(p):
    return pl.pallas_call(
        _combine_body,
        grid=(NB,),
        in_specs=[pl.BlockSpec((2, BN, H), lambda i: (0, i, 0))],
        out_specs=pl.BlockSpec((BN, H), lambda i: (i, 0)),
        out_shape=jax.ShapeDtypeStruct((NP, H), _F32),
    )(p)


def _edge_agg_body(gidx_hbm, dst_hbm, hw_hbm, zeros_hbm, out_hbm,
                   idx_v, dst_v, rows, agg, *sems):
    gsem, ssem = sems[:RBUF], sems[RBUF:]
    cid = lax.axis_index("c")
    sid = lax.axis_index("s")
    my_rows = pl.multiple_of(sid * ROWS_PER_SUB, 8)

    pltpu.sync_copy(zeros_hbm, agg.at[pl.ds(my_rows, ROWS_PER_SUB)])

    def run(base_chunk, k):
        pltpu.sync_copy(gidx_hbm.at[pl.ds(base_chunk, k)],
                        idx_v.at[pl.ds(0, k)])
        pltpu.sync_copy(dst_hbm.at[pl.ds(base_chunk, k)],
                        dst_v.at[pl.ds(0, k)])
        plsc.subcore_barrier()
        for c in range(DEPTH):
            pltpu.async_copy(hw_hbm.at[idx_v.at[c]], rows.at[c % RBUF],
                             gsem[c % RBUF])

        def group(jj, carry):
            for kk in range(RBUF):
                c = jj * RBUF + kk

                @pl.when(c < k)
                def _():
                    pltpu.make_async_copy(hw_hbm.at[idx_v.at[c]],
                                          rows.at[kk], gsem[kk]).wait()
                    pltpu.async_copy(rows.at[kk], agg.at[dst_v.at[c]],
                                     ssem[kk], add=True)
                    n = c + DEPTH
                    kn = (kk + DEPTH) % RBUF

                    @pl.when(n < k)
                    def _():
                        @pl.when(n >= RBUF)
                        def _():
                            pltpu.make_async_copy(
                                rows.at[kn], agg.at[dst_v.at[n - RBUF]],
                                ssem[kn]).wait()
                        pltpu.async_copy(hw_hbm.at[idx_v.at[n]],
                                         rows.at[kn], gsem[kn])
            return carry

        lax.fori_loop(0, (k + RBUF - 1) // RBUF, group, 0)
        for c in range(k - RBUF, k):
            pltpu.make_async_copy(rows.at[c % RBUF], agg.at[dst_v.at[c]],
                                  ssem[c % RBUF]).wait()

    @pl.when(cid == 0)
    def _():
        run(pl.multiple_of(sid * CHUNKS_SID, 8), K0)

    @pl.when(cid == 1)
    def _():
        run(sid * CHUNKS_SID + K0, K1)

    plsc.subcore_barrier()
    off = pl.multiple_of(cid * NP + sid * ROWS_PER_SUB, 8)
    pltpu.sync_copy(agg.at[pl.ds(my_rows, ROWS_PER_SUB)],
                    out_hbm.at[pl.ds(off, ROWS_PER_SUB)])


def _edge_agg(gidx, dst, hw_flat, zeros):
    mesh = plsc.VectorSubcoreMesh(core_axis_name="c", subcore_axis_name="s",
                                  num_cores=NCORES, num_subcores=NSUB)
    k = pl.kernel(
        _edge_agg_body,
        out_type=jax.ShapeDtypeStruct((NCORES * NP, H), _BF16),
        mesh=mesh,
        scratch_types=[
            pltpu.VMEM((K0, C), jnp.int32),
            pltpu.VMEM((K0, C), jnp.int32),
            pltpu.VMEM((RBUF, C, H), _BF16),
            pltpu.VMEM_SHARED((NP, H), _BF16),
        ] + [pltpu.SemaphoreType.DMA] * (2 * RBUF),
        compiler_params=pltpu.CompilerParams(use_tc_tiling_on_sc=False),
    )
    return k(gidx, dst, hw_flat, zeros)


def kernel(class_objects, states_objects, edge_tuples, edge_classes,
           mask_object, mask_edge, class_emb, state_W, state_b,
           W1, b1, W2, b2, rgcn_W0, rgcn_W1):
    src = edge_tuples[:, :, 0].astype(jnp.int32)
    dst = edge_tuples[:, :, 1].astype(jnp.int32)
    rel = edge_classes.astype(jnp.int32)
    gidx = ((src // BN) * NR + rel) * BN + src % BN
    gidx_pad = jnp.concatenate(
        [gidx, jnp.zeros((B, EPB - E), jnp.int32)], axis=1)
    dst_pad = jnp.concatenate(
        [dst, jnp.full((B, EPB - E), DUMMY_DST, jnp.int32)], axis=1)
    gidx_pad = gidx_pad.reshape(B, NSUB * CHUNKS_SID, C)
    dst_pad = dst_pad.reshape(B, NSUB * CHUNKS_SID, C)

    cls_pad = jnp.pad(class_objects.astype(_F32), ((0, 0), (0, NP - N)))
    cls_pad = cls_pad[:, :, None]
    states_pad = jnp.pad(states_objects, ((0, 0), (0, NP - N), (0, 0)))

    swt = state_W.T
    sb = state_b[None, :]
    w1t, w2t = W1.T, W2.T
    b1r, b2r = b1[None, :], b2[None, :]
    wcat0 = rgcn_W0.transpose(1, 0, 2).reshape(H, NR * H)
    wcat1 = rgcn_W1.transpose(1, 0, 2).reshape(H, NR * H)
    zeros = jnp.zeros((ROWS_PER_SUB, H), _BF16)

    outs = []
    for b in range(B):
        hw1 = _mlp_einsum(cls_pad[b], states_pad[b], class_emb, swt, sb,
                          w1t, b1r, w2t, b2r, wcat0)
        p1 = _edge_agg(gidx_pad[b], dst_pad[b], hw1, zeros)
        hw2 = _combine_einsum(p1.reshape(NCORES, NP, H), wcat1)
        p2 = _edge_agg(gidx_pad[b], dst_pad[b], hw2, zeros)
        outs.append(_combine(p2.reshape(NCORES, NP, H)))
    return jnp.stack(outs, axis=0)[:, :N, :]


# speedup vs baseline: 2.6411x; 1.0294x over previous
import jax
import jax.numpy as jnp
from jax import lax
from jax.experimental import pallas as pl
from jax.experimental.pallas import tpu as pltpu
from jax.experimental.pallas import tpu_sc as plsc

B = 4
N = 12500
NP = 12800
E = 200000
EPB = 204800
H = 64
H2 = 32
NS = 30
NC = 300
NR = 16

BN = 1280
NB = NP // BN

NCORES = 2
NSUB = 16
C = 80
CHUNKS_SID = 160
K0 = 125
K1 = CHUNKS_SID - K0
ROWS_PER_SUB = NP // NSUB
DUMMY_DST = 12600

RBUF = 10
DEPTH = 5

_F32 = jnp.float32
_BF16 = jnp.bfloat16


def _mlp_einsum_body(cls_ref, st_ref, cemb_ref, swt_ref, sb_ref,
                     w1t_ref, b1_ref, w2t_ref, b2_ref, wcat_ref, out_ref):
    cls = cls_ref[:, :].astype(jnp.int32)
    iota = lax.broadcasted_iota(jnp.int32, (BN, NC), 1)
    onehot = (iota == cls).astype(_F32)
    cn = jnp.dot(onehot, cemb_ref[...], preferred_element_type=_F32)
    se = jnp.dot(st_ref[...], swt_ref[...], preferred_element_type=_F32)
    se = se + sb_ref[...]
    h = jax.nn.relu(jnp.concatenate([cn, se], axis=1))
    h = jax.nn.relu(jnp.dot(h, w1t_ref[...], preferred_element_type=_F32)
                    + b1_ref[...])
    h = jax.nn.relu(jnp.dot(h, w2t_ref[...], preferred_element_type=_F32)
                    + b2_ref[...])
    hw = jnp.dot(h, wcat_ref[...],
                 preferred_element_type=_F32).astype(_BF16)
    for r in range(NR):
        out_ref[r * BN:(r + 1) * BN, :] = hw[:, r * H:(r + 1) * H]


def _mlp_einsum(cls_b, states_b, class_emb, swt, sb, w1t, b1, w2t, b2, wcat):
    full = lambda shape: pl.BlockSpec(shape, lambda i: (0,) * len(shape))
    return pl.pallas_call(
        _mlp_einsum_body,
        grid=(NB,),
        in_specs=[
            pl.BlockSpec((BN, 1), lambda i: (i, 0)),
            pl.BlockSpec((BN, NS), lambda i: (i, 0)),
            full((NC, H2)), full((NS, H2)), full((1, H2)),
            full((H, H)), full((1, H)), full((H, H)), full((1, H)),
            full((H, NR * H)),
        ],
        out_specs=pl.BlockSpec((NR * BN, H), lambda i: (i, 0)),
        out_shape=jax.ShapeDtypeStruct((NP * NR, H), _BF16),
    )(cls_b, states_b, class_emb, swt, sb, w1t, b1, w2t, b2, wcat)


def _combine_einsum_body(p_ref, wcat_ref, out_ref):
    h = jax.nn.relu(p_ref[0].astype(_F32) + p_ref[1].astype(_F32))
    hw = jnp.dot(h, wcat_ref[...],
                 preferred_element_type=_F32).astype(_BF16)
    for r in range(NR):
        out_ref[r * BN:(r + 1) * BN, :] = hw[:, r * H:(r + 1) * H]


def _combine_einsum(p, wcat):
    return pl.pallas_call(
        _combine_einsum_body,
        grid=(NB,),
        in_specs=[
            pl.BlockSpec((2, BN, H), lambda i: (0, i, 0)),
            pl.BlockSpec((H, NR * H), lambda i: (0, 0)),
        ],
        out_specs=pl.BlockSpec((NR * BN, H), lambda i: (i, 0)),
        out_shape=jax.ShapeDtypeStruct((NP * NR, H), _BF16),
    )(p, wcat)


def _combine_body(p_ref, out_ref):
    out_ref[...] = jax.nn.relu(p_ref[0].astype(_F32) + p_ref[1].astype(_F32))


def _combine(p):
    return pl.pallas_call(
        _combine_body,
        grid=(NB,),
        in_specs=[pl.BlockSpec((2, BN, H), lambda i: (0, i, 0))],
        out_specs=pl.BlockSpec((BN, H), lambda i: (i, 0)),
        out_shape=jax.ShapeDtypeStruct((NP, H), _F32),
    )(p)


def _edge_agg_body(b, gidx_hbm, dst_hbm, hw_hbm, zeros_hbm, out_hbm,
                   idx_v, dst_v, rows, agg, *sems):
    gsem, ssem = sems[:RBUF], sems[RBUF:]
    cid = lax.axis_index("c")
    sid = lax.axis_index("s")
    my_rows = pl.multiple_of(sid * ROWS_PER_SUB, 8)

    pltpu.sync_copy(zeros_hbm, agg.at[pl.ds(my_rows, ROWS_PER_SUB)])

    def run(base_chunk, k):
        pltpu.sync_copy(gidx_hbm.at[pl.ds(base_chunk, k)],
                        idx_v.at[pl.ds(0, k)])
        pltpu.sync_copy(dst_hbm.at[pl.ds(base_chunk, k)],
                        dst_v.at[pl.ds(0, k)])
        plsc.subcore_barrier()
        for c in range(DEPTH):
            pltpu.async_copy(hw_hbm.at[idx_v.at[c]], rows.at[c % RBUF],
                             gsem[c % RBUF])

        def group(jj, carry):
            for kk in range(RBUF):
                c = jj * RBUF + kk

                @pl.when(c < k)
                def _():
                    pltpu.make_async_copy(hw_hbm.at[idx_v.at[c]],
                                          rows.at[kk], gsem[kk]).wait()
                    pltpu.async_copy(rows.at[kk], agg.at[dst_v.at[c]],
                                     ssem[kk], add=True)
                    n = c + DEPTH
                    kn = (kk + DEPTH) % RBUF

                    @pl.when(n < k)
                    def _():
                        @pl.when(n >= RBUF)
                        def _():
                            pltpu.make_async_copy(
                                rows.at[kn], agg.at[dst_v.at[n - RBUF]],
                                ssem[kn]).wait()
                        pltpu.async_copy(hw_hbm.at[idx_v.at[n]],
                                         rows.at[kn], gsem[kn])
            return carry

        lax.fori_loop(0, (k + RBUF - 1) // RBUF, group, 0)
        for c in range(k - RBUF, k):
            pltpu.make_async_copy(rows.at[c % RBUF], agg.at[dst_v.at[c]],
                                  ssem[c % RBUF]).wait()

    bbase = b * NSUB * CHUNKS_SID

    @pl.when(cid == 0)
    def _():
        run(bbase + sid * CHUNKS_SID, K0)

    @pl.when(cid == 1)
    def _():
        run(bbase + sid * CHUNKS_SID + K0, K1)

    plsc.subcore_barrier()
    off = pl.multiple_of(cid * NP + sid * ROWS_PER_SUB, 8)
    pltpu.sync_copy(agg.at[pl.ds(my_rows, ROWS_PER_SUB)],
                    out_hbm.at[pl.ds(off, ROWS_PER_SUB)])


def _edge_agg(gidx, dst, hw_flat, zeros, b):
    import functools
    mesh = plsc.VectorSubcoreMesh(core_axis_name="c", subcore_axis_name="s",
                                  num_cores=NCORES, num_subcores=NSUB)
    k = pl.kernel(
        functools.partial(_edge_agg_body, b),
        out_type=jax.ShapeDtypeStruct((NCORES * NP, H), _BF16),
        mesh=mesh,
        scratch_types=[
            pltpu.VMEM((K0, C), jnp.int32),
            pltpu.VMEM((K0, C), jnp.int32),
            pltpu.VMEM((RBUF, C, H), _BF16),
            pltpu.VMEM_SHARED((NP, H), _BF16),
        ] + [pltpu.SemaphoreType.DMA] * (2 * RBUF),
        compiler_params=pltpu.CompilerParams(use_tc_tiling_on_sc=False),
    )
    return k(gidx, dst, hw_flat, zeros)


def kernel(class_objects, states_objects, edge_tuples, edge_classes,
           mask_object, mask_edge, class_emb, state_W, state_b,
           W1, b1, W2, b2, rgcn_W0, rgcn_W1):
    src = edge_tuples[:, :, 0].astype(jnp.int32)
    dst = edge_tuples[:, :, 1].astype(jnp.int32)
    rel = edge_classes.astype(jnp.int32)
    gidx = ((src // BN) * NR + rel) * BN + src % BN
    gidx_pad = jnp.concatenate(
        [gidx, jnp.zeros((B, EPB - E), jnp.int32)], axis=1)
    dst_pad = jnp.concatenate(
        [dst, jnp.full((B, EPB - E), DUMMY_DST, jnp.int32)], axis=1)
    gidx_pad = gidx_pad.reshape(B * NSUB * CHUNKS_SID, C)
    dst_pad = dst_pad.reshape(B * NSUB * CHUNKS_SID, C)

    cls_pad = jnp.pad(class_objects.astype(_F32), ((0, 0), (0, NP - N)))
    cls_pad = cls_pad[:, :, None]
    states_pad = jnp.pad(states_objects, ((0, 0), (0, NP - N), (0, 0)))

    swt = state_W.T
    sb = state_b[None, :]
    w1t, w2t = W1.T, W2.T
    b1r, b2r = b1[None, :], b2[None, :]
    wcat0 = rgcn_W0.transpose(1, 0, 2).reshape(H, NR * H)
    wcat1 = rgcn_W1.transpose(1, 0, 2).reshape(H, NR * H)
    zeros = jnp.zeros((ROWS_PER_SUB, H), _BF16)

    outs = []
    for b in range(B):
        hw1 = _mlp_einsum(cls_pad[b], states_pad[b], class_emb, swt, sb,
                          w1t, b1r, w2t, b2r, wcat0)
        p1 = _edge_agg(gidx_pad, dst_pad, hw1, zeros, b)
        hw2 = _combine_einsum(p1.reshape(NCORES, NP, H), wcat1)
        p2 = _edge_agg(gidx_pad, dst_pad, hw2, zeros, b)
        outs.append(_combine(p2.reshape(NCORES, NP, H)))
    return jnp.stack(outs, axis=0)[:, :N, :]
